# Initial kernel scaffold; baseline (speedup 1.0000x reference)
import jax
import jax.numpy as jnp
from jax.experimental import pallas as pl


def kernel(x, batch, query, Wk, Wv):
    raise NotImplementedError("write your pallas kernel here")


# speedup vs baseline: 14.0962x; 14.0962x over previous
import jax
import jax.numpy as jnp
from jax.experimental import pallas as pl
from jax.experimental.pallas import tpu as pltpu

N = 50000
D = 512
G = 256
B = 1000
NB = N // B


def _body(batch_ref, x_ref, q_ref, wk_ref, wv_ref, out_ref,
          m_ref, s_ref, acc_ref):
    i = pl.program_id(0)
    neg = jnp.float32(-jnp.inf)

    @pl.when(i == 0)
    def _init():
        m_ref[...] = jnp.full((G, 1), neg, jnp.float32)
        s_ref[...] = jnp.zeros((G, 1), jnp.float32)
        acc_ref[...] = jnp.zeros((G, D), jnp.float32)

    x = x_ref[...]
    b = batch_ref[0]
    k = jax.lax.dot_general(
        x, wk_ref[...], (((1,), (1,)), ((), ())),
        preferred_element_type=jnp.float32)
    scores = jax.lax.dot_general(
        q_ref[...], k, (((1,), (1,)), ((), ())),
        preferred_element_type=jnp.float32)
    gids = jax.lax.broadcasted_iota(jnp.int32, (G, B), 0)
    oh = gids == b

    m_blk = jnp.max(jnp.where(oh, scores, neg), axis=1, keepdims=True)
    m_old = m_ref[...]
    m_new = jnp.maximum(m_old, m_blk)
    scale = jnp.where(m_new == neg, 0.0, jnp.exp(m_old - m_new))
    m_row = jnp.max(jnp.where(oh, m_new, neg), axis=0, keepdims=True)
    e = jnp.exp(scores - m_row)
    w = jnp.where(oh, e, 0.0)
    s_ref[...] = s_ref[...] * scale + jnp.sum(w, axis=1, keepdims=True)
    acc_ref[...] = acc_ref[...] * scale + jax.lax.dot_general(
        w, x, (((1,), (0,)), ((), ())), preferred_element_type=jnp.float32)
    m_ref[...] = m_new

    @pl.when(i == NB - 1)
    def _fin():
        pooled = acc_ref[...] / (s_ref[...] + 1e-16)
        out_ref[...] = jax.lax.dot_general(
            pooled, wv_ref[...], (((1,), (1,)), ((), ())),
            preferred_element_type=jnp.float32,
            precision=jax.lax.Precision.HIGHEST)


def kernel(x, batch, query, Wk, Wv):
    b3 = batch.reshape(NB, 1, B)
    q2 = query.reshape(1, D)
    return pl.pallas_call(
        _body,
        grid=(NB,),
        in_specs=[
            pl.BlockSpec((1, 1, B), lambda i: (i, 0, 0)),
            pl.BlockSpec((B, D), lambda i: (i, 0)),
            pl.BlockSpec((1, D), lambda i: (0, 0)),
            pl.BlockSpec((D, D), lambda i: (0, 0)),
            pl.BlockSpec((D, D), lambda i: (0, 0)),
        ],
        out_specs=pl.BlockSpec((G, D), lambda i: (0, 0)),
        out_shape=jax.ShapeDtypeStruct((G, D), jnp.float32),
        scratch_shapes=[
            pltpu.VMEM((G, 1), jnp.float32),
            pltpu.VMEM((G, 1), jnp.float32),
            pltpu.VMEM((G, D), jnp.float32),
        ],
    )(b3, x, q2, Wk, Wv)
